# baseline (speedup 1.0000x reference)
import jax
import jax.numpy as jnp
import numpy as np
from jax.experimental import pallas as pl
from jax.experimental.pallas import tpu as pltpu

H = 16
D = 64
E = H * D
BQ = 512
KC = 512


def _fused_attn_kernel(q_ref, k_ref, v_ref, w_ref, b_ref, o_ref):
    q = q_ref[0]
    k = k_ref[0]
    v = v_ref[0]
    n = k.shape[0]
    bd = (jax.lax.broadcasted_iota(jnp.int32, (E, H), 0) // D
          == jax.lax.broadcasted_iota(jnp.int32, (E, H), 1)
          ).astype(jnp.float32)
    qf = q.astype(jnp.float32)
    kf = k.astype(jnp.float32)
    qn2 = jax.lax.dot_general(
        qf * qf, bd, (((1,), (0,)), ((), ())),
        preferred_element_type=jnp.float32)
    kn2 = jax.lax.dot_general(
        kf * kf, bd, (((1,), (0,)), ((), ())),
        preferred_element_type=jnp.float32)
    kn2m = jnp.max(kn2, axis=0, keepdims=True)
    shifts = jnp.sqrt(qn2) * (jnp.sqrt(kn2m) * 1.02)
    acc = jnp.broadcast_to(b_ref[...], (BQ, D)).astype(jnp.float32)
    for h in range(H):
        sl = slice(h * D, (h + 1) * D)
        qh = q[:, sl]
        kh = k[:, sl]
        vh = v[:, sl]
        shift = shifts[:, h:h + 1]
        vaug = jnp.concatenate(
            [vh, jnp.ones((n, D), jnp.bfloat16)], axis=1)
        oh_full = jnp.zeros((BQ, 2 * D), jnp.float32)
        for kc in range(0, n, KC):
            ks = slice(kc, kc + KC)
            s = jax.lax.dot_general(
                qh, kh[ks], (((1,), (1,)), ((), ())),
                preferred_element_type=jnp.float32)
            e = jnp.exp2(s - shift).astype(jnp.bfloat16)
            oh_full = oh_full + jax.lax.dot_general(
                e, vaug[ks], (((1,), (0,)), ((), ())),
                preferred_element_type=jnp.float32)
        oh = oh_full[:, :D] / oh_full[:, D:D + 1]
        wh = w_ref[:, sl]
        acc = acc + jax.lax.dot_general(
            oh, wh, (((1,), (1,)), ((), ())),
            preferred_element_type=jnp.float32)
    o_ref[0] = acc


@jax.jit
def kernel(query, key, value, W_out, b_out):
    b, n, e = query.shape
    scale = np.log2(np.e) / np.sqrt(D)
    qb = (query * scale).astype(jnp.bfloat16)
    kb = key.astype(jnp.bfloat16)
    vb = value.astype(jnp.bfloat16)
    grid = (b, n // BQ)
    out = pl.pallas_call(
        _fused_attn_kernel,
        grid=grid,
        in_specs=[
            pl.BlockSpec((1, BQ, e), lambda bi, qi: (bi, qi, 0)),
            pl.BlockSpec((1, n, e), lambda bi, qi: (bi, 0, 0)),
            pl.BlockSpec((1, n, e), lambda bi, qi: (bi, 0, 0)),
            pl.BlockSpec((D, e), lambda bi, qi: (0, 0)),
            pl.BlockSpec((1, D), lambda bi, qi: (0, 0)),
        ],
        out_specs=pl.BlockSpec((1, BQ, D), lambda bi, qi: (bi, qi, 0)),
        out_shape=jax.ShapeDtypeStruct((b, n, D), jnp.float32),
        compiler_params=pltpu.CompilerParams(
            dimension_semantics=("parallel", "parallel"),
        ),
    )(qb, kb, vb, W_out, b_out.reshape(1, D))
    return out


# speedup vs baseline: 1.0282x; 1.0282x over previous
import jax
import jax.numpy as jnp
import numpy as np
from jax.experimental import pallas as pl
from jax.experimental.pallas import tpu as pltpu

H = 16
D = 64
E = H * D
BQ = 512


def _fused_attn_kernel(q_ref, k_ref, v_ref, w_ref, b_ref, o_ref):
    q = q_ref[0]
    k = k_ref[0]
    v = v_ref[0]
    n = k.shape[0]
    bd = (jax.lax.broadcasted_iota(jnp.int32, (E, H), 0) // D
          == jax.lax.broadcasted_iota(jnp.int32, (E, H), 1)
          ).astype(jnp.float32)
    qf = q.astype(jnp.float32)
    kf = k.astype(jnp.float32)
    qn2 = jax.lax.dot_general(
        qf * qf, bd, (((1,), (0,)), ((), ())),
        preferred_element_type=jnp.float32)
    kn2 = jax.lax.dot_general(
        kf * kf, bd, (((1,), (0,)), ((), ())),
        preferred_element_type=jnp.float32)
    kn2m = jnp.max(kn2, axis=0, keepdims=True)
    shifts = jnp.sqrt(qn2) * (jnp.sqrt(kn2m) * 1.02)
    acc = jnp.broadcast_to(b_ref[...], (BQ, D)).astype(jnp.float32)
    for h in range(H):
        sl = slice(h * D, (h + 1) * D)
        qh = q[:, sl]
        kh = k[:, sl]
        vh = v[:, sl]
        qh_aug = jnp.concatenate(
            [qh, shifts[:, h:h + 1].astype(jnp.bfloat16)], axis=1)
        kh_aug = jnp.concatenate(
            [kh, jnp.full((n, 1), -1.0, jnp.bfloat16)], axis=1)
        s = jax.lax.dot_general(
            qh_aug, kh_aug, (((1,), (1,)), ((), ())),
            preferred_element_type=jnp.float32)
        e = jnp.exp2(s).astype(jnp.bfloat16)
        vaug = jnp.concatenate(
            [vh, jnp.ones((n, D), jnp.bfloat16)], axis=1)
        oh_full = jax.lax.dot_general(
            e, vaug, (((1,), (0,)), ((), ())),
            preferred_element_type=jnp.float32)
        oh = oh_full[:, :D] / oh_full[:, D:D + 1]
        wh = w_ref[:, sl]
        acc = acc + jax.lax.dot_general(
            oh, wh, (((1,), (1,)), ((), ())),
            preferred_element_type=jnp.float32)
    o_ref[0] = acc


@jax.jit
def kernel(query, key, value, W_out, b_out):
    b, n, e = query.shape
    scale = np.log2(np.e) / np.sqrt(D)
    qb = (query * scale).astype(jnp.bfloat16)
    kb = key.astype(jnp.bfloat16)
    vb = value.astype(jnp.bfloat16)
    grid = (b, n // BQ)
    out = pl.pallas_call(
        _fused_attn_kernel,
        grid=grid,
        in_specs=[
            pl.BlockSpec((1, BQ, e), lambda bi, qi: (bi, qi, 0)),
            pl.BlockSpec((1, n, e), lambda bi, qi: (bi, 0, 0)),
            pl.BlockSpec((1, n, e), lambda bi, qi: (bi, 0, 0)),
            pl.BlockSpec((D, e), lambda bi, qi: (0, 0)),
            pl.BlockSpec((1, D), lambda bi, qi: (0, 0)),
        ],
        out_specs=pl.BlockSpec((1, BQ, D), lambda bi, qi: (bi, qi, 0)),
        out_shape=jax.ShapeDtypeStruct((b, n, D), jnp.float32),
        compiler_params=pltpu.CompilerParams(
            dimension_semantics=("parallel", "parallel"),
        ),
    )(qb, kb, vb, W_out, b_out.reshape(1, D))
    return out


# speedup vs baseline: 1.0495x; 1.0207x over previous
import jax
import jax.numpy as jnp
import numpy as np
from jax.experimental import pallas as pl
from jax.experimental.pallas import tpu as pltpu

H = 16
D = 64
E = H * D
BQ = 512


def _fused_attn_kernel(q_ref, k_ref, v_ref, w_ref, b_ref, o_ref):
    q = q_ref[0]
    k = k_ref[0]
    v = v_ref[0]
    n = k.shape[0]
    bd = (jax.lax.broadcasted_iota(jnp.int32, (E, H), 0) // D
          == jax.lax.broadcasted_iota(jnp.int32, (E, H), 1)
          ).astype(jnp.float32)
    qf = q.astype(jnp.float32)
    kf = k.astype(jnp.float32)
    qn2 = jax.lax.dot_general(
        qf * qf, bd, (((1,), (0,)), ((), ())),
        preferred_element_type=jnp.float32)
    kn2 = jax.lax.dot_general(
        kf * kf, bd, (((1,), (0,)), ((), ())),
        preferred_element_type=jnp.float32)
    kn2m = jnp.max(kn2, axis=0, keepdims=True)
    shifts = jnp.sqrt(qn2) * (jnp.sqrt(kn2m) * 1.02)
    acc = jnp.broadcast_to(b_ref[...], (BQ, D)).astype(jnp.float32)
    for h in range(H):
        sl = slice(h * D, (h + 1) * D)
        qh = q[:, sl]
        kh = k[:, sl]
        vh = v[:, sl]
        s = jax.lax.dot_general(
            qh, kh, (((1,), (1,)), ((), ())),
            preferred_element_type=jnp.float32)
        e = jnp.exp2(s - shifts[:, h:h + 1]).astype(jnp.bfloat16)
        vaug = jnp.concatenate(
            [vh, jnp.ones((n, D), jnp.bfloat16)], axis=1)
        oh_full = jax.lax.dot_general(
            e, vaug, (((1,), (0,)), ((), ())),
            preferred_element_type=jnp.float32)
        oh = oh_full[:, :D] / oh_full[:, D:D + 1]
        wh = w_ref[:, sl]
        acc = acc + jax.lax.dot_general(
            oh, wh, (((1,), (1,)), ((), ())),
            preferred_element_type=jnp.float32)
    o_ref[0] = acc


@jax.jit
def kernel(query, key, value, W_out, b_out):
    b, n, e = query.shape
    scale = np.log2(np.e) / np.sqrt(D)
    qb = (query * scale).astype(jnp.bfloat16)
    kb = key.astype(jnp.bfloat16)
    vb = value.astype(jnp.bfloat16)
    grid = (b, n // BQ)
    out = pl.pallas_call(
        _fused_attn_kernel,
        grid=grid,
        in_specs=[
            pl.BlockSpec((1, BQ, e), lambda bi, qi: (bi, qi, 0)),
            pl.BlockSpec((1, n, e), lambda bi, qi: (bi, 0, 0)),
            pl.BlockSpec((1, n, e), lambda bi, qi: (bi, 0, 0)),
            pl.BlockSpec((D, e), lambda bi, qi: (0, 0)),
            pl.BlockSpec((1, D), lambda bi, qi: (0, 0)),
        ],
        out_specs=pl.BlockSpec((1, BQ, D), lambda bi, qi: (bi, qi, 0)),
        out_shape=jax.ShapeDtypeStruct((b, n, D), jnp.float32),
        compiler_params=pltpu.CompilerParams(
            dimension_semantics=("parallel", "parallel"),
        ),
    )(qb, kb, vb, W_out, b_out.reshape(1, D))
    return out
